# Initial kernel scaffold; baseline (speedup 1.0000x reference)
import jax
import jax.numpy as jnp
from jax.experimental import pallas as pl


def kernel(edge_attr, edge_index, node_batch, Gamma_W, Gamma_b, Lambda_W):
    raise NotImplementedError("write your pallas kernel here")


# speedup vs baseline: 11.4351x; 11.4351x over previous
import jax
import jax.numpy as jnp
from jax.experimental import pallas as pl
from jax.experimental.pallas import tpu as pltpu

_G = 64
_B = 2560

_INTERPRET = False


def kernel(edge_attr, edge_index, node_batch, Gamma_W, Gamma_b, Lambda_W):
    E, D = edge_attr.shape
    G, B = _G, _B
    NB = E // B
    assert E % B == 0

    src = edge_index[0].astype(jnp.int32)
    nb32 = node_batch.astype(jnp.int32)
    starts = jnp.searchsorted(
        nb32, jnp.arange(G + 1, dtype=jnp.int32), side="left"
    ).astype(jnp.int32)
    smat = jnp.broadcast_to(starts[:G, None], (G, B))
    emat = jnp.broadcast_to(starts[1:, None], (G, B))
    src_r = src.reshape(NB, 1, B)
    lamT = Lambda_W.T
    gamT_bf = Gamma_W.T.astype(jnp.bfloat16)
    gb = Gamma_b.reshape(1, D)

    def _pool_body(src_ref, ea_ref, smat_ref, emat_ref, lamT_ref, gb_ref,
                   padj_ref, acc_ref):
        i = pl.program_id(0)

        @pl.when(i == 0)
        def _():
            acc_ref[...] = jnp.zeros_like(acc_ref)

        srcb = jnp.broadcast_to(src_ref[0], (G, B))
        ind = (srcb >= smat_ref[...]) & (srcb < emat_ref[...])
        indT = ind.astype(jnp.bfloat16)
        ea = ea_ref[...].astype(jnp.bfloat16)
        acc_ref[...] += jax.lax.dot_general(
            indT, ea, (((1,), (0,)), ((), ())),
            preferred_element_type=jnp.float32)

        @pl.when(i == NB - 1)
        def _():
            padj_ref[...] = jax.lax.dot_general(
                acc_ref[...], lamT_ref[...], (((1,), (0,)), ((), ())),
                preferred_element_type=jnp.float32) - gb_ref[...]

    padj = pl.pallas_call(
        _pool_body,
        grid=(NB,),
        in_specs=[
            pl.BlockSpec((1, 1, B), lambda i: (i, 0, 0)),
            pl.BlockSpec((B, D), lambda i: (i, 0)),
            pl.BlockSpec((G, B), lambda i: (0, 0)),
            pl.BlockSpec((G, B), lambda i: (0, 0)),
            pl.BlockSpec((D, D), lambda i: (0, 0)),
            pl.BlockSpec((1, D), lambda i: (0, 0)),
        ],
        out_specs=pl.BlockSpec((G, D), lambda i: (0, 0)),
        out_shape=jax.ShapeDtypeStruct((G, D), jnp.float32),
        scratch_shapes=[pltpu.VMEM((G, D), jnp.float32)],
        interpret=_INTERPRET,
    )(src_r, edge_attr, smat, emat, lamT, gb)

    def _out_body(src_ref, ea_ref, smat_ref, emat_ref, gamT_ref, padj_ref,
                  out_ref):
        srcb = jnp.broadcast_to(src_ref[0], (G, B))
        ind = (srcb >= smat_ref[...]) & (srcb < emat_ref[...])
        dense = jax.lax.dot_general(
            ea_ref[...].astype(jnp.bfloat16), gamT_ref[...],
            (((1,), (0,)), ((), ())),
            preferred_element_type=jnp.float32)
        unpool = jax.lax.dot_general(
            ind.astype(jnp.float32), padj_ref[...], (((0,), (0,)), ((), ())),
            preferred_element_type=jnp.float32)
        out_ref[...] = jnp.maximum(dense - unpool, 0.0)

    out = pl.pallas_call(
        _out_body,
        grid=(NB,),
        in_specs=[
            pl.BlockSpec((1, 1, B), lambda i: (i, 0, 0)),
            pl.BlockSpec((B, D), lambda i: (i, 0)),
            pl.BlockSpec((G, B), lambda i: (0, 0)),
            pl.BlockSpec((G, B), lambda i: (0, 0)),
            pl.BlockSpec((D, D), lambda i: (0, 0)),
            pl.BlockSpec((G, D), lambda i: (0, 0)),
        ],
        out_specs=pl.BlockSpec((B, D), lambda i: (i, 0)),
        out_shape=jax.ShapeDtypeStruct((E, D), jnp.float32),
        interpret=_INTERPRET,
    )(src_r, edge_attr, smat, emat, gamT_bf, padj)
    return out
